# Initial kernel scaffold; baseline (speedup 1.0000x reference)
import jax
import jax.numpy as jnp
from jax.experimental import pallas as pl


def kernel(x, word_emb):
    raise NotImplementedError("write your pallas kernel here")


# speedup vs baseline: 1.2268x; 1.2268x over previous
import functools

import jax
import jax.numpy as jnp
from jax import lax
from jax.experimental import pallas as pl
from jax.experimental.pallas import tpu as pltpu
from jax.experimental.pallas import tpu_sc as plsc

MODEL_DIM = 512
CHUNK = 80


@functools.lru_cache(maxsize=None)
def _make_gather(B, D):
    info = plsc.get_sparse_core_info()
    NC, NS = info.num_cores, info.num_subcores
    NW = NC * NS
    assert B % (NW * CHUNK) == 0
    b_per_w = B // NW
    n_chunks = b_per_w // CHUNK
    mesh = plsc.VectorSubcoreMesh(core_axis_name="c", subcore_axis_name="s")

    @functools.partial(
        pl.kernel,
        mesh=mesh,
        out_type=jax.ShapeDtypeStruct((B, D), jnp.float32),
        scratch_types=[
            pltpu.VMEM((b_per_w,), jnp.int32),
            pltpu.VMEM((CHUNK, D), jnp.float32),
            pltpu.SemaphoreType.DMA,
        ],
    )
    def gather_kernel(idx_hbm, table_hbm, out_hbm, idx_v, rows_v, sem):
        wid = lax.axis_index("s") * NC + lax.axis_index("c")
        base = wid * b_per_w
        pltpu.sync_copy(idx_hbm.at[pl.ds(base, b_per_w)], idx_v)

        def body(c, carry):
            off = c * CHUNK
            pltpu.async_copy(
                table_hbm.at[idx_v.at[pl.ds(off, CHUNK)]], rows_v, sem
            ).wait()
            pltpu.sync_copy(rows_v, out_hbm.at[pl.ds(base + off, CHUNK)])
            return carry

        lax.fori_loop(0, n_chunks, body, 0)

    return gather_kernel


@jax.jit
def kernel(x, word_emb):
    B, S = x.shape
    D = word_emb.shape[1]
    idx = x.reshape(-1).astype(jnp.int32)
    out = _make_gather(B * S, D)(idx, word_emb)
    return out.reshape(B, S, D)


# speedup vs baseline: 1.2828x; 1.0457x over previous
import functools

import jax
import jax.numpy as jnp
from jax import lax
from jax.experimental import pallas as pl
from jax.experimental.pallas import tpu as pltpu
from jax.experimental.pallas import tpu_sc as plsc

CHUNK = 80


@functools.lru_cache(maxsize=None)
def _make_gather(B, D):
    info = plsc.get_sparse_core_info()
    NC, NS = info.num_cores, info.num_subcores
    NW = NC * NS
    assert B % (NW * CHUNK) == 0
    b_per_w = B // NW
    n_chunks = b_per_w // CHUNK
    assert n_chunks % 2 == 0 and n_chunks >= 4
    mesh = plsc.VectorSubcoreMesh(core_axis_name="c", subcore_axis_name="s")

    @functools.partial(
        pl.kernel,
        mesh=mesh,
        out_type=jax.ShapeDtypeStruct((B, D), jnp.float32),
        scratch_types=[
            pltpu.VMEM((b_per_w,), jnp.int32),
            pltpu.VMEM((CHUNK, D), jnp.float32),
            pltpu.VMEM((CHUNK, D), jnp.float32),
            pltpu.SemaphoreType.DMA,
            pltpu.SemaphoreType.DMA,
            pltpu.SemaphoreType.DMA,
            pltpu.SemaphoreType.DMA,
        ],
    )
    def gather_kernel(idx_hbm, table_hbm, out_hbm, idx_v, buf0, buf1,
                      sg0, sg1, so0, so1):
        wid = lax.axis_index("s") * NC + lax.axis_index("c")
        base = wid * b_per_w
        bufs = (buf0, buf1)
        sgs = (sg0, sg1)
        sos = (so0, so1)

        def start_gather(c, b):
            pltpu.async_copy(
                table_hbm.at[idx_v.at[pl.ds(c * CHUNK, CHUNK)]], bufs[b], sgs[b])

        def wait_gather(b):
            pltpu.make_async_copy(
                table_hbm.at[idx_v.at[pl.ds(0, CHUNK)]], bufs[b], sgs[b]).wait()

        def start_out(c, b):
            pltpu.async_copy(
                bufs[b], out_hbm.at[pl.ds(base + c * CHUNK, CHUNK)], sos[b])

        def wait_out(c, b):
            pltpu.make_async_copy(
                bufs[b], out_hbm.at[pl.ds(base + c * CHUNK, CHUNK)], sos[b]).wait()

        pltpu.sync_copy(idx_hbm.at[pl.ds(base, b_per_w)], idx_v)

        start_gather(0, 0)
        wait_gather(0)
        start_out(0, 0)
        start_gather(1, 1)

        def body(i, carry):
            c = 1 + 2 * i
            for b, cc in ((1, c), (0, c + 1)):
                wait_gather(b)
                start_out(cc, b)
                wait_out(cc - 1, b ^ 1)
                start_gather(cc + 1, b ^ 1)
            return carry

        lax.fori_loop(0, (n_chunks - 2) // 2, body, 0)

        wait_gather(1)
        start_out(n_chunks - 1, 1)
        wait_out(n_chunks - 2, 0)
        wait_out(n_chunks - 1, 1)

    return gather_kernel


@jax.jit
def kernel(x, word_emb):
    B, S = x.shape
    D = word_emb.shape[1]
    idx = x.reshape(-1).astype(jnp.int32)
    out = _make_gather(B * S, D)(idx, word_emb)
    return out.reshape(B, S, D)


# speedup vs baseline: 1.8117x; 1.4123x over previous
import functools

import jax
import jax.numpy as jnp
from jax import lax
from jax.experimental import pallas as pl
from jax.experimental.pallas import tpu as pltpu
from jax.experimental.pallas import tpu_sc as plsc

ROWS_PER_CHUNK = 2


@functools.lru_cache(maxsize=None)
def _make_gather(B, S, D):
    info = plsc.get_sparse_core_info()
    NC, NS = info.num_cores, info.num_subcores
    NW = NC * NS
    assert B % (NW * ROWS_PER_CHUNK) == 0
    rows_per_w = B // NW
    n_chunks = rows_per_w // ROWS_PER_CHUNK
    assert n_chunks % 2 == 0 and n_chunks >= 4
    mesh = plsc.VectorSubcoreMesh(core_axis_name="c", subcore_axis_name="s")

    @functools.partial(
        pl.kernel,
        mesh=mesh,
        out_type=jax.ShapeDtypeStruct((B, S, D), jnp.float32),
        scratch_types=[
            pltpu.VMEM((rows_per_w, S), jnp.int32),
            pltpu.VMEM((ROWS_PER_CHUNK, S, D), jnp.float32),
            pltpu.VMEM((ROWS_PER_CHUNK, S, D), jnp.float32),
            pltpu.SemaphoreType.DMA,
            pltpu.SemaphoreType.DMA,
            pltpu.SemaphoreType.DMA,
            pltpu.SemaphoreType.DMA,
        ],
    )
    def gather_kernel(x_hbm, table_hbm, out_hbm, idx_v, buf0, buf1,
                      sg0, sg1, so0, so1):
        wid = lax.axis_index("s") * NC + lax.axis_index("c")
        base = wid * rows_per_w
        bufs = (buf0, buf1)
        sgs = (sg0, sg1)
        sos = (so0, so1)

        def start_gather(c, b):
            for j in range(ROWS_PER_CHUNK):
                pltpu.async_copy(
                    table_hbm.at[idx_v.at[c * ROWS_PER_CHUNK + j]],
                    bufs[b].at[j], sgs[b])

        def wait_gather(b):
            for j in range(ROWS_PER_CHUNK):
                pltpu.make_async_copy(
                    table_hbm.at[idx_v.at[j]], bufs[b].at[j], sgs[b]).wait()

        def start_out(c, b):
            pltpu.async_copy(
                bufs[b],
                out_hbm.at[pl.ds(base + c * ROWS_PER_CHUNK, ROWS_PER_CHUNK)],
                sos[b])

        def wait_out(c, b):
            pltpu.make_async_copy(
                bufs[b],
                out_hbm.at[pl.ds(base + c * ROWS_PER_CHUNK, ROWS_PER_CHUNK)],
                sos[b]).wait()

        pltpu.sync_copy(x_hbm.at[pl.ds(base, rows_per_w)], idx_v)

        start_gather(0, 0)
        wait_gather(0)
        start_out(0, 0)
        start_gather(1, 1)

        def body(i, carry):
            c = 1 + 2 * i
            for b, cc in ((1, c), (0, c + 1)):
                wait_gather(b)
                start_out(cc, b)
                wait_out(cc - 1, b ^ 1)
                start_gather(cc + 1, b ^ 1)
            return carry

        lax.fori_loop(0, (n_chunks - 2) // 2, body, 0)

        wait_gather(1)
        start_out(n_chunks - 1, 1)
        wait_out(n_chunks - 2, 0)
        wait_out(n_chunks - 1, 1)

    return gather_kernel


@jax.jit
def kernel(x, word_emb):
    B, S = x.shape
    D = word_emb.shape[1]
    return _make_gather(B, S, D)(x.astype(jnp.int32), word_emb)


# speedup vs baseline: 1.8415x; 1.0165x over previous
import functools

import jax
import jax.numpy as jnp
from jax import lax
from jax.experimental import pallas as pl
from jax.experimental.pallas import tpu as pltpu
from jax.experimental.pallas import tpu_sc as plsc


@functools.lru_cache(maxsize=None)
def _make_gather(B, S, D):
    info = plsc.get_sparse_core_info()
    NC, NS, L = info.num_cores, info.num_subcores, info.num_lanes
    NW = NC * NS
    assert L == 16 and D % 128 == 0 and B % (8 * NW) == 0
    DT = D // 128
    b_per_w = B // NW
    TL = b_per_w // 8
    Q = B * DT
    q_per_w = TL * DT * 8
    assert q_per_w == 128, "one 128-entry gather per (worker, s)"
    assert S % 2 == 0
    mesh = plsc.VectorSubcoreMesh(core_axis_name="c", subcore_axis_name="s")

    @functools.partial(
        pl.kernel,
        mesh=mesh,
        out_type=jax.ShapeDtypeStruct((S, Q, 128), jnp.float32),
        compiler_params=pltpu.CompilerParams(needs_layout_passes=False),
        scratch_types=[
            pltpu.VMEM((b_per_w, S), jnp.int32),
            pltpu.VMEM((q_per_w,), jnp.int32),
            pltpu.VMEM((q_per_w,), jnp.int32),
            pltpu.VMEM((q_per_w, 128), jnp.float32),
            pltpu.VMEM((q_per_w, 128), jnp.float32),
            pltpu.SemaphoreType.DMA,
            pltpu.SemaphoreType.DMA,
            pltpu.SemaphoreType.DMA,
            pltpu.SemaphoreType.DMA,
        ],
    )
    def gather_kernel(x_hbm, table_hbm, out_hbm, x_v, ib0, ib1, db0, db1,
                      sg0, sg1, so0, so1):
        wid = lax.axis_index("s") * NC + lax.axis_index("c")
        ibs = (ib0, ib1)
        dbs = (db0, db1)
        sgs = (sg0, sg1)
        sos = (so0, so1)

        lane = lax.iota(jnp.int32, L)
        rb = lax.bitwise_and(lane, 7)
        hi = lax.shift_right_logical(lane, 3)

        def compute_idx(s, b):
            col = jnp.full((L,), s, dtype=jnp.int32)
            for h in range(q_per_w // L):
                rows = 8 * (h // 2) + rb
                vals = plsc.load_gather(x_v, [rows, col])
                idx = 4 * vals + (2 * (h % 2) + hi)
                ibs[b][pl.ds(h * L, L)] = idx

        def start_gather(b):
            pltpu.async_copy(table_hbm.at[ibs[b]], dbs[b], sgs[b])

        def wait_gather(b):
            pltpu.make_async_copy(table_hbm.at[ibs[b]], dbs[b], sgs[b]).wait()

        def start_out(s, b):
            pltpu.async_copy(
                dbs[b], out_hbm.at[s, pl.ds(q_per_w * wid, q_per_w)], sos[b])

        def wait_out(s, b):
            pltpu.make_async_copy(
                dbs[b], out_hbm.at[s, pl.ds(q_per_w * wid, q_per_w)],
                sos[b]).wait()

        pltpu.sync_copy(x_hbm.at[pl.ds(wid * b_per_w, b_per_w)], x_v)

        compute_idx(0, 0)
        start_gather(0)
        wait_gather(0)
        start_out(0, 0)
        compute_idx(1, 1)
        start_gather(1)

        def body(i, carry):
            s = 1 + 2 * i
            for b, ss in ((1, s), (0, s + 1)):
                wait_gather(b)
                start_out(ss, b)
                wait_out(ss - 1, b ^ 1)
                compute_idx(ss + 1, b ^ 1)
                start_gather(b ^ 1)
            return carry

        lax.fori_loop(0, (S - 2) // 2, body, 0)

        wait_gather(1)
        start_out(S - 1, 1)
        wait_out(S - 2, 0)
        wait_out(S - 1, 1)

    return gather_kernel


@jax.jit
def kernel(x, word_emb):
    B, S = x.shape
    V, D = word_emb.shape
    table4 = word_emb.reshape(V * (D // 128), 128)
    out5 = _make_gather(B, S, D)(x.astype(jnp.int32), table4)
    out = out5.reshape(S, B // 8, D // 128, 8, 128)
    out = out.transpose(1, 3, 0, 2, 4).reshape(B, S, D)
    return out
